# baseline (speedup 1.0000x reference)
import functools

import jax
import jax.numpy as jnp
from jax import lax
from jax.experimental import pallas as pl
from jax.experimental.pallas import tpu as pltpu
from jax.experimental.pallas import tpu_sc as plsc

NC = 2
NS = 16
NW = NC * NS
CHUNK = 128


def _leaky(t):
    return jnp.where(t >= 0, t, 0.01 * t)


def _dot(a, b):
    return jnp.dot(a, b, preferred_element_type=jnp.float32)


def _bias_pack(biases):
    rows = jnp.stack(biases, axis=0)
    return jnp.pad(rows, ((0, 8 - rows.shape[0]), (0, 0)))


@functools.lru_cache(maxsize=None)
def _gather_fn(n, f, e_pad, k):
    ew = e_pad // NW
    mesh = plsc.VectorSubcoreMesh(core_axis_name="c", subcore_axis_name="s")

    @functools.partial(
        pl.kernel,
        mesh=mesh,
        out_type=jax.ShapeDtypeStruct((e_pad, f), jnp.float32),
        scratch_types=[
            pltpu.VMEM((k, CHUNK), jnp.int32),
            pltpu.VMEM((CHUNK, f), jnp.float32),
            pltpu.SemaphoreType.DMA,
        ],
    )
    def gather(table_hbm, idx_hbm, out_hbm, idx_v, rows_v, gsem):
        wid = lax.axis_index("s") * NC + lax.axis_index("c")
        base = wid * ew
        pltpu.sync_copy(idx_hbm.at[wid], idx_v)

        @pl.loop(0, k)
        def _(j):
            pltpu.async_copy(table_hbm.at[idx_v.at[j]], rows_v, gsem).wait()
            pltpu.sync_copy(rows_v, out_hbm.at[pl.ds(base + j * CHUNK, CHUNK)])

    return gather


@functools.lru_cache(maxsize=None)
def _scatter_fn(n, e_pad, k):
    ew = e_pad // NW
    zr = -(-(-(-n // NS)) // 8) * 8
    n_pad = NS * zr
    zfull, zrem = zr // CHUNK, zr % CHUNK
    mesh = plsc.VectorSubcoreMesh(core_axis_name="c", subcore_axis_name="s")

    def _zero_acc(rows_v, acc_sh, sid):
        zero = jnp.zeros((16,), jnp.float32)

        @pl.loop(0, CHUNK)
        def _(r):
            for c8 in range(8):
                rows_v[r, pl.ds(c8 * 16, 16)] = zero

        zb = sid * zr
        for t in range(zfull):
            pltpu.sync_copy(rows_v, acc_sh.at[pl.ds(zb + t * CHUNK, CHUNK)])
        if zrem:
            pltpu.sync_copy(
                rows_v.at[pl.ds(0, zrem)],
                acc_sh.at[pl.ds(zb + zfull * CHUNK, zrem)],
            )

    def _write_acc(rows_v, acc_sh, out_hbm, cid, sid):
        zb = sid * zr
        ob = cid * n_pad + zb
        for t in range(zfull):
            pltpu.sync_copy(acc_sh.at[pl.ds(zb + t * CHUNK, CHUNK)], rows_v)
            pltpu.sync_copy(rows_v, out_hbm.at[pl.ds(ob + t * CHUNK, CHUNK)])
        if zrem:
            pltpu.sync_copy(
                acc_sh.at[pl.ds(zb + zfull * CHUNK, zrem)],
                rows_v.at[pl.ds(0, zrem)],
            )
            pltpu.sync_copy(
                rows_v.at[pl.ds(0, zrem)],
                out_hbm.at[pl.ds(ob + zfull * CHUNK, zrem)],
            )

    @functools.partial(
        pl.kernel,
        mesh=mesh,
        out_type=jax.ShapeDtypeStruct((NC * n_pad, 128), jnp.float32),
        scratch_types=[
            pltpu.VMEM((k, CHUNK), jnp.int32),
            pltpu.VMEM((CHUNK, 128), jnp.float32),
            pltpu.VMEM_SHARED((n_pad, 128), jnp.float32),
            pltpu.SemaphoreType.DMA,
        ],
    )
    def scatter(vals_hbm, idx_hbm, out_hbm, idx_v, rows_v, acc_sh, sem):
        cid = lax.axis_index("c")
        sid = lax.axis_index("s")
        wid = sid * NC + cid
        base = wid * ew

        _zero_acc(rows_v, acc_sh, sid)
        plsc.subcore_barrier()

        pltpu.sync_copy(idx_hbm.at[wid], idx_v)

        @pl.loop(0, k)
        def _(j):
            pltpu.sync_copy(vals_hbm.at[pl.ds(base + j * CHUNK, CHUNK)], rows_v)
            pltpu.sync_copy(rows_v, acc_sh.at[idx_v.at[j]], add=True)

        plsc.subcore_barrier()
        _write_acc(rows_v, acc_sh, out_hbm, cid, sid)

    return scatter


@functools.lru_cache(maxsize=None)
def _counts_fn(n, e_pad, k, e_num):
    ew = e_pad // NW
    zr = -(-(-(-n // NS)) // 8) * 8
    n_pad = NS * zr
    zfull, zrem = zr // CHUNK, zr % CHUNK
    mesh = plsc.VectorSubcoreMesh(core_axis_name="c", subcore_axis_name="s")

    @functools.partial(
        pl.kernel,
        mesh=mesh,
        out_type=jax.ShapeDtypeStruct((NC * n_pad, 128), jnp.float32),
        scratch_types=[
            pltpu.VMEM((k, CHUNK), jnp.int32),
            pltpu.VMEM((2, CHUNK, 128), jnp.float32),
            pltpu.VMEM_SHARED((n_pad, 128), jnp.float32),
            pltpu.SemaphoreType.DMA,
        ],
    )
    def counts(idx_hbm, out_hbm, idx_v, rows_v, acc_sh, sem):
        cid = lax.axis_index("c")
        sid = lax.axis_index("s")
        wid = sid * NC + cid
        base = wid * ew
        n_real = jnp.clip(e_num - base, 0, ew)
        kf = n_real // CHUNK
        prem = n_real % CHUNK

        zero = jnp.zeros((16,), jnp.float32)

        @pl.loop(0, CHUNK)
        def _(r):
            for c8 in range(8):
                rows_v[0, r, pl.ds(c8 * 16, 16)] = zero
                rows_v[1, r, pl.ds(c8 * 16, 16)] = jnp.where(
                    r < prem, 1.0, 0.0
                ) * jnp.ones((16,), jnp.float32)

        zb = sid * zr
        for t in range(zfull):
            pltpu.sync_copy(rows_v.at[0], acc_sh.at[pl.ds(zb + t * CHUNK, CHUNK)])
        if zrem:
            pltpu.sync_copy(
                rows_v.at[0, pl.ds(0, zrem)],
                acc_sh.at[pl.ds(zb + zfull * CHUNK, zrem)],
            )
        plsc.subcore_barrier()

        pltpu.sync_copy(idx_hbm.at[wid], idx_v)

        @pl.loop(0, CHUNK)
        def _(r):
            for c8 in range(8):
                rows_v[0, r, pl.ds(c8 * 16, 16)] = jnp.ones((16,), jnp.float32)

        @pl.loop(0, kf)
        def _(j):
            pltpu.sync_copy(rows_v.at[0], acc_sh.at[idx_v.at[j]], add=True)

        @pl.when(prem > 0)
        def _():
            pltpu.sync_copy(rows_v.at[1], acc_sh.at[idx_v.at[kf]], add=True)

        plsc.subcore_barrier()

        ob = cid * n_pad + zb
        for t in range(zfull):
            pltpu.sync_copy(acc_sh.at[pl.ds(zb + t * CHUNK, CHUNK)], rows_v.at[0])
            pltpu.sync_copy(rows_v.at[0], out_hbm.at[pl.ds(ob + t * CHUNK, CHUNK)])
        if zrem:
            pltpu.sync_copy(
                acc_sh.at[pl.ds(zb + zfull * CHUNK, zrem)],
                rows_v.at[0, pl.ds(0, zrem)],
            )
            pltpu.sync_copy(
                rows_v.at[0, pl.ds(0, zrem)],
                out_hbm.at[pl.ds(ob + zfull * CHUNK, zrem)],
            )

    return counts


def _node_pre(h, wpack, bpack):
    n = h.shape[0]
    bn = 2000

    def body(h_ref, w_ref, b_ref, hac_ref, hb_ref):
        hh = h_ref[...]
        hac_ref[:, 0:128] = _dot(hh, w_ref[0:128]) + b_ref[0:1, :]
        hac_ref[:, 128:256] = _dot(hh, w_ref[256:384]) + b_ref[1:2, :]
        hb_ref[...] = _dot(hh, w_ref[128:256])

    return pl.pallas_call(
        body,
        grid=(n // bn,),
        in_specs=[
            pl.BlockSpec((bn, 128), lambda i: (i, 0)),
            pl.BlockSpec((384, 128), lambda i: (0, 0)),
            pl.BlockSpec((8, 128), lambda i: (0, 0)),
        ],
        out_specs=[
            pl.BlockSpec((bn, 256), lambda i: (i, 0)),
            pl.BlockSpec((bn, 128), lambda i: (i, 0)),
        ],
        out_shape=[
            jax.ShapeDtypeStruct((n, 256), jnp.float32),
            jax.ShapeDtypeStruct((n, 128), jnp.float32),
        ],
        compiler_params=pltpu.CompilerParams(dimension_semantics=("parallel",)),
    )(h, wpack, bpack)


def _edge_mlps(gac, gb, e, wpack, bpack, e_real):
    e_pad = e.shape[0]
    be = 1024

    def body(gac_ref, gb_ref, e_ref, w_ref, b_ref, enew_ref, m_ref):
        i = pl.program_id(0)
        u = _leaky(gac_ref[:, 0:128] + gb_ref[...] + _dot(e_ref[...], w_ref[0:128]))
        u = _leaky(_dot(u, w_ref[128:256]) + b_ref[0:1, :])
        en = _dot(u, w_ref[256:384]) + b_ref[1:2, :]
        enew_ref[...] = en
        v = _leaky(gac_ref[:, 128:256] + _dot(en, w_ref[384:512]))
        v = _leaky(_dot(v, w_ref[512:640]) + b_ref[2:3, :])
        m = _dot(v, w_ref[640:768]) + b_ref[3:4, :]
        rowid = i * be + lax.broadcasted_iota(jnp.int32, (be, 1), 0)
        m_ref[...] = jnp.where(rowid < e_real, m, 0.0)

    blk = pl.BlockSpec((be, 128), lambda i: (i, 0))
    osh = jax.ShapeDtypeStruct((e_pad, 128), jnp.float32)
    return pl.pallas_call(
        body,
        grid=(e_pad // be,),
        in_specs=[
            pl.BlockSpec((be, 256), lambda i: (i, 0)),
            blk, blk,
            pl.BlockSpec((768, 128), lambda i: (0, 0)),
            pl.BlockSpec((8, 128), lambda i: (0, 0)),
        ],
        out_specs=[blk, blk],
        out_shape=[osh, osh],
        compiler_params=pltpu.CompilerParams(dimension_semantics=("parallel",)),
    )(gac, gb, e, wpack, bpack)


def _node_update(h, s0, s1, c0, c1, wpack, bpack):
    n = h.shape[0]
    bn = 2000

    def body(h_ref, s0_ref, s1_ref, c0_ref, c1_ref, w_ref, b_ref, out_ref):
        cnt = jnp.maximum(c0_ref[...] + c1_ref[...], 1.0)
        agg = (s0_ref[...] + s1_ref[...]) / cnt
        t = _leaky(
            _dot(h_ref[...], w_ref[0:128]) + _dot(agg, w_ref[128:256]) + b_ref[0:1, :]
        )
        t = _leaky(_dot(t, w_ref[256:384]) + b_ref[1:2, :])
        out_ref[...] = _dot(t, w_ref[384:512]) + b_ref[2:3, :]

    blk = pl.BlockSpec((bn, 128), lambda i: (i, 0))
    return pl.pallas_call(
        body,
        grid=(n // bn,),
        in_specs=[
            blk, blk, blk, blk, blk,
            pl.BlockSpec((512, 128), lambda i: (0, 0)),
            pl.BlockSpec((8, 128), lambda i: (0, 0)),
        ],
        out_specs=blk,
        out_shape=jax.ShapeDtypeStruct((n, 128), jnp.float32),
        compiler_params=pltpu.CompilerParams(dimension_semantics=("parallel",)),
    )(h, s0, s1, c0, c1, wpack, bpack)


def kernel(x, edge_index, edge_attr, params):
    n, d = x.shape
    e_num = edge_attr.shape[0]
    k = -(-e_num // (NW * CHUNK))
    e_pad = NW * CHUNK * k
    pad = e_pad - e_num

    row = edge_index[0].astype(jnp.int32)
    col = edge_index[1].astype(jnp.int32)
    row3 = jnp.pad(row, (0, pad)).reshape(NW, k, CHUNK)
    col3 = jnp.pad(col, (0, pad)).reshape(NW, k, CHUNK)
    e = jnp.pad(edge_attr, ((0, pad), (0, 0)))

    n_pad = NS * (-(-(-(-n // NS)) // 8) * 8)
    spread = max(n_pad - n, 1)
    pad_idx = n_pad - 1 - (jnp.arange(pad, dtype=jnp.int32) % spread)
    col3s = jnp.concatenate([col, pad_idx]).reshape(NW, k, CHUNK)

    cnt2 = _counts_fn(n, e_pad, k, e_num)(col3s)
    c0, c1 = cnt2[:n], cnt2[n_pad:n_pad + n]

    h = x
    for lp in params:
        (w1, b1), (w2, b2), (w3, b3) = lp["edge"]
        (v1, cb1), (v2, cb2), (v3, cb3) = lp["node1"]
        (u1, d1), (u2, d2), (u3, d3) = lp["node2"]

        wpre = jnp.concatenate([w1[0:128], w1[128:256], v1[0:128]], axis=0)
        hac, hb = _node_pre(h, wpre, _bias_pack([b1, cb1]))

        gac = _gather_fn(n, 256, e_pad, k)(hac, row3)
        gb = _gather_fn(n, 128, e_pad, k)(hb, col3)

        wedge = jnp.concatenate([w1[256:384], w2, w3, v1[128:256], v2, v3], axis=0)
        e, m = _edge_mlps(gac, gb, e, wedge, _bias_pack([b2, b3, cb2, cb3]), e_num)

        s2 = _scatter_fn(n, e_pad, k)(m, col3s)

        wn2 = jnp.concatenate([u1[0:128], u1[128:256], u2, u3], axis=0)
        h = _node_update(h, s2[:n], s2[n_pad:n_pad + n], c0, c1, wn2,
                         _bias_pack([d1, d2, d3]))

    return h


# speedup vs baseline: 1.0154x; 1.0154x over previous
import functools

import jax
import jax.numpy as jnp
from jax import lax
from jax.experimental import pallas as pl
from jax.experimental.pallas import tpu as pltpu
from jax.experimental.pallas import tpu_sc as plsc

NC = 2
NS = 16
NW = NC * NS
CHUNK = 128


def _leaky(t):
    return jnp.where(t >= 0, t, 0.01 * t)


def _dot(a, b):
    return jnp.dot(a, b, preferred_element_type=jnp.float32)


def _bias_pack(biases):
    rows = jnp.stack(biases, axis=0)
    return jnp.pad(rows, ((0, 8 - rows.shape[0]), (0, 0)))


@functools.lru_cache(maxsize=None)
def _gather_fn(n, f, e_pad, k):
    ew = e_pad // NW
    mesh = plsc.VectorSubcoreMesh(core_axis_name="c", subcore_axis_name="s")

    @functools.partial(
        pl.kernel,
        mesh=mesh,
        out_type=jax.ShapeDtypeStruct((e_pad, f), jnp.float32),
        scratch_types=[
            pltpu.VMEM((k, CHUNK), jnp.int32),
            pltpu.VMEM((CHUNK, f), jnp.float32),
            pltpu.SemaphoreType.DMA,
        ],
    )
    def gather(table_hbm, idx_hbm, out_hbm, idx_v, rows_v, gsem):
        wid = lax.axis_index("s") * NC + lax.axis_index("c")
        base = wid * ew
        pltpu.sync_copy(idx_hbm.at[wid], idx_v)

        @pl.loop(0, k)
        def _(j):
            pltpu.async_copy(table_hbm.at[idx_v.at[j]], rows_v, gsem).wait()
            pltpu.sync_copy(rows_v, out_hbm.at[pl.ds(base + j * CHUNK, CHUNK)])

    return gather


@functools.lru_cache(maxsize=None)
def _scatter_fn(n, e_pad, k):
    ew = e_pad // NW
    zr = -(-(-(-n // NS)) // 8) * 8
    n_pad = NS * zr
    zfull, zrem = zr // CHUNK, zr % CHUNK
    mesh = plsc.VectorSubcoreMesh(core_axis_name="c", subcore_axis_name="s")

    def _zero_acc(rows_v, acc_sh, sid):
        zero = jnp.zeros((16,), jnp.float32)

        @pl.loop(0, CHUNK)
        def _(r):
            for c8 in range(8):
                rows_v[r, pl.ds(c8 * 16, 16)] = zero

        zb = sid * zr
        for t in range(zfull):
            pltpu.sync_copy(rows_v, acc_sh.at[pl.ds(zb + t * CHUNK, CHUNK)])
        if zrem:
            pltpu.sync_copy(
                rows_v.at[pl.ds(0, zrem)],
                acc_sh.at[pl.ds(zb + zfull * CHUNK, zrem)],
            )

    def _write_acc(rows_v, acc_sh, out_hbm, cid, sid):
        zb = sid * zr
        ob = cid * n_pad + zb
        for t in range(zfull):
            pltpu.sync_copy(acc_sh.at[pl.ds(zb + t * CHUNK, CHUNK)], rows_v)
            pltpu.sync_copy(rows_v, out_hbm.at[pl.ds(ob + t * CHUNK, CHUNK)])
        if zrem:
            pltpu.sync_copy(
                acc_sh.at[pl.ds(zb + zfull * CHUNK, zrem)],
                rows_v.at[pl.ds(0, zrem)],
            )
            pltpu.sync_copy(
                rows_v.at[pl.ds(0, zrem)],
                out_hbm.at[pl.ds(ob + zfull * CHUNK, zrem)],
            )

    @functools.partial(
        pl.kernel,
        mesh=mesh,
        out_type=jax.ShapeDtypeStruct((NC * n_pad, 128), jnp.float32),
        scratch_types=[
            pltpu.VMEM((k, CHUNK), jnp.int32),
            pltpu.VMEM((2, CHUNK, 128), jnp.float32),
            pltpu.VMEM_SHARED((n_pad, 128), jnp.float32),
            pltpu.SemaphoreType.DMA,
            pltpu.SemaphoreType.DMA,
            pltpu.SemaphoreType.DMA,
            pltpu.SemaphoreType.DMA,
        ],
    )
    def scatter(vals_hbm, idx_hbm, out_hbm, idx_v, rows_v, acc_sh,
                ls0, ls1, ss0, ss1):
        cid = lax.axis_index("c")
        sid = lax.axis_index("s")
        wid = sid * NC + cid
        base = wid * ew

        _zero_acc(rows_v.at[0], acc_sh, sid)
        plsc.subcore_barrier()

        pltpu.sync_copy(idx_hbm.at[wid], idx_v)

        @pl.loop(0, k - k % 2, step=2)
        def _(j0):
            for b in range(2):
                j = j0 + b
                lsem = ls0 if b == 0 else ls1
                ssem = ss0 if b == 0 else ss1
                buf = rows_v.at[b]
                src = vals_hbm.at[pl.ds(base + j * CHUNK, CHUNK)]

                @pl.when(j >= 2)
                def _():
                    pltpu.make_async_copy(buf, acc_sh.at[idx_v.at[j]], ssem).wait()

                pltpu.async_copy(src, buf, lsem)
                pltpu.make_async_copy(src, buf, lsem).wait()
                pltpu.async_copy(buf, acc_sh.at[idx_v.at[j]], ssem, add=True)

        for b in range(2):
            ssem = ss0 if b == 0 else ss1
            pltpu.make_async_copy(
                rows_v.at[b], acc_sh.at[idx_v.at[b]], ssem
            ).wait()

        if k % 2:
            pltpu.sync_copy(
                vals_hbm.at[pl.ds(base + (k - 1) * CHUNK, CHUNK)], rows_v.at[0]
            )
            pltpu.sync_copy(rows_v.at[0], acc_sh.at[idx_v.at[k - 1]], add=True)

        plsc.subcore_barrier()
        _write_acc(rows_v.at[0], acc_sh, out_hbm, cid, sid)

    return scatter


@functools.lru_cache(maxsize=None)
def _counts_fn(n, e_pad, k, e_num):
    ew = e_pad // NW
    zr = -(-(-(-n // NS)) // 8) * 8
    n_pad = NS * zr
    zfull, zrem = zr // CHUNK, zr % CHUNK
    mesh = plsc.VectorSubcoreMesh(core_axis_name="c", subcore_axis_name="s")

    @functools.partial(
        pl.kernel,
        mesh=mesh,
        out_type=jax.ShapeDtypeStruct((NC * n_pad, 128), jnp.float32),
        scratch_types=[
            pltpu.VMEM((k, CHUNK), jnp.int32),
            pltpu.VMEM((2, CHUNK, 128), jnp.float32),
            pltpu.VMEM_SHARED((n_pad, 128), jnp.float32),
            pltpu.SemaphoreType.DMA,
        ],
    )
    def counts(idx_hbm, out_hbm, idx_v, rows_v, acc_sh, sem):
        cid = lax.axis_index("c")
        sid = lax.axis_index("s")
        wid = sid * NC + cid
        base = wid * ew
        n_real = jnp.clip(e_num - base, 0, ew)
        kf = n_real // CHUNK
        prem = n_real % CHUNK

        zero = jnp.zeros((16,), jnp.float32)

        @pl.loop(0, CHUNK)
        def _(r):
            for c8 in range(8):
                rows_v[0, r, pl.ds(c8 * 16, 16)] = zero
                rows_v[1, r, pl.ds(c8 * 16, 16)] = jnp.where(
                    r < prem, 1.0, 0.0
                ) * jnp.ones((16,), jnp.float32)

        zb = sid * zr
        for t in range(zfull):
            pltpu.sync_copy(rows_v.at[0], acc_sh.at[pl.ds(zb + t * CHUNK, CHUNK)])
        if zrem:
            pltpu.sync_copy(
                rows_v.at[0, pl.ds(0, zrem)],
                acc_sh.at[pl.ds(zb + zfull * CHUNK, zrem)],
            )
        plsc.subcore_barrier()

        pltpu.sync_copy(idx_hbm.at[wid], idx_v)

        @pl.loop(0, CHUNK)
        def _(r):
            for c8 in range(8):
                rows_v[0, r, pl.ds(c8 * 16, 16)] = jnp.ones((16,), jnp.float32)

        @pl.loop(0, kf)
        def _(j):
            pltpu.sync_copy(rows_v.at[0], acc_sh.at[idx_v.at[j]], add=True)

        @pl.when(prem > 0)
        def _():
            pltpu.sync_copy(rows_v.at[1], acc_sh.at[idx_v.at[kf]], add=True)

        plsc.subcore_barrier()

        ob = cid * n_pad + zb
        for t in range(zfull):
            pltpu.sync_copy(acc_sh.at[pl.ds(zb + t * CHUNK, CHUNK)], rows_v.at[0])
            pltpu.sync_copy(rows_v.at[0], out_hbm.at[pl.ds(ob + t * CHUNK, CHUNK)])
        if zrem:
            pltpu.sync_copy(
                acc_sh.at[pl.ds(zb + zfull * CHUNK, zrem)],
                rows_v.at[0, pl.ds(0, zrem)],
            )
            pltpu.sync_copy(
                rows_v.at[0, pl.ds(0, zrem)],
                out_hbm.at[pl.ds(ob + zfull * CHUNK, zrem)],
            )

    return counts


def _node_pre(h, wpack, bpack):
    n = h.shape[0]
    bn = 2000

    def body(h_ref, w_ref, b_ref, hac_ref, hb_ref):
        hh = h_ref[...]
        hac_ref[:, 0:128] = _dot(hh, w_ref[0:128]) + b_ref[0:1, :]
        hac_ref[:, 128:256] = _dot(hh, w_ref[256:384]) + b_ref[1:2, :]
        hb_ref[...] = _dot(hh, w_ref[128:256])

    return pl.pallas_call(
        body,
        grid=(n // bn,),
        in_specs=[
            pl.BlockSpec((bn, 128), lambda i: (i, 0)),
            pl.BlockSpec((384, 128), lambda i: (0, 0)),
            pl.BlockSpec((8, 128), lambda i: (0, 0)),
        ],
        out_specs=[
            pl.BlockSpec((bn, 256), lambda i: (i, 0)),
            pl.BlockSpec((bn, 128), lambda i: (i, 0)),
        ],
        out_shape=[
            jax.ShapeDtypeStruct((n, 256), jnp.float32),
            jax.ShapeDtypeStruct((n, 128), jnp.float32),
        ],
        compiler_params=pltpu.CompilerParams(dimension_semantics=("parallel",)),
    )(h, wpack, bpack)


def _edge_mlps(gac, gb, e, wpack, bpack, e_real):
    e_pad = e.shape[0]
    be = 1024

    def body(gac_ref, gb_ref, e_ref, w_ref, b_ref, enew_ref, m_ref):
        i = pl.program_id(0)
        u = _leaky(gac_ref[:, 0:128] + gb_ref[...] + _dot(e_ref[...], w_ref[0:128]))
        u = _leaky(_dot(u, w_ref[128:256]) + b_ref[0:1, :])
        en = _dot(u, w_ref[256:384]) + b_ref[1:2, :]
        enew_ref[...] = en
        v = _leaky(gac_ref[:, 128:256] + _dot(en, w_ref[384:512]))
        v = _leaky(_dot(v, w_ref[512:640]) + b_ref[2:3, :])
        m = _dot(v, w_ref[640:768]) + b_ref[3:4, :]
        rowid = i * be + lax.broadcasted_iota(jnp.int32, (be, 1), 0)
        m_ref[...] = jnp.where(rowid < e_real, m, 0.0)

    blk = pl.BlockSpec((be, 128), lambda i: (i, 0))
    osh = jax.ShapeDtypeStruct((e_pad, 128), jnp.float32)
    return pl.pallas_call(
        body,
        grid=(e_pad // be,),
        in_specs=[
            pl.BlockSpec((be, 256), lambda i: (i, 0)),
            blk, blk,
            pl.BlockSpec((768, 128), lambda i: (0, 0)),
            pl.BlockSpec((8, 128), lambda i: (0, 0)),
        ],
        out_specs=[blk, blk],
        out_shape=[osh, osh],
        compiler_params=pltpu.CompilerParams(dimension_semantics=("parallel",)),
    )(gac, gb, e, wpack, bpack)


def _node_update(h, s0, s1, c0, c1, wpack, bpack):
    n = h.shape[0]
    bn = 2000

    def body(h_ref, s0_ref, s1_ref, c0_ref, c1_ref, w_ref, b_ref, out_ref):
        cnt = jnp.maximum(c0_ref[...] + c1_ref[...], 1.0)
        agg = (s0_ref[...] + s1_ref[...]) / cnt
        t = _leaky(
            _dot(h_ref[...], w_ref[0:128]) + _dot(agg, w_ref[128:256]) + b_ref[0:1, :]
        )
        t = _leaky(_dot(t, w_ref[256:384]) + b_ref[1:2, :])
        out_ref[...] = _dot(t, w_ref[384:512]) + b_ref[2:3, :]

    blk = pl.BlockSpec((bn, 128), lambda i: (i, 0))
    return pl.pallas_call(
        body,
        grid=(n // bn,),
        in_specs=[
            blk, blk, blk, blk, blk,
            pl.BlockSpec((512, 128), lambda i: (0, 0)),
            pl.BlockSpec((8, 128), lambda i: (0, 0)),
        ],
        out_specs=blk,
        out_shape=jax.ShapeDtypeStruct((n, 128), jnp.float32),
        compiler_params=pltpu.CompilerParams(dimension_semantics=("parallel",)),
    )(h, s0, s1, c0, c1, wpack, bpack)


def kernel(x, edge_index, edge_attr, params):
    n, d = x.shape
    e_num = edge_attr.shape[0]
    k = -(-e_num // (NW * CHUNK))
    e_pad = NW * CHUNK * k
    pad = e_pad - e_num

    row = edge_index[0].astype(jnp.int32)
    col = edge_index[1].astype(jnp.int32)
    row3 = jnp.pad(row, (0, pad)).reshape(NW, k, CHUNK)
    col3 = jnp.pad(col, (0, pad)).reshape(NW, k, CHUNK)
    e = jnp.pad(edge_attr, ((0, pad), (0, 0)))

    n_pad = NS * (-(-(-(-n // NS)) // 8) * 8)
    spread = max(n_pad - n, 1)
    pad_idx = n_pad - 1 - (jnp.arange(pad, dtype=jnp.int32) % spread)
    col3s = jnp.concatenate([col, pad_idx]).reshape(NW, k, CHUNK)

    cnt2 = _counts_fn(n, e_pad, k, e_num)(col3s)
    c0, c1 = cnt2[:n], cnt2[n_pad:n_pad + n]

    h = x
    for lp in params:
        (w1, b1), (w2, b2), (w3, b3) = lp["edge"]
        (v1, cb1), (v2, cb2), (v3, cb3) = lp["node1"]
        (u1, d1), (u2, d2), (u3, d3) = lp["node2"]

        wpre = jnp.concatenate([w1[0:128], w1[128:256], v1[0:128]], axis=0)
        hac, hb = _node_pre(h, wpre, _bias_pack([b1, cb1]))

        gac = _gather_fn(n, 256, e_pad, k)(hac, row3)
        gb = _gather_fn(n, 128, e_pad, k)(hb, col3)

        wedge = jnp.concatenate([w1[256:384], w2, w3, v1[128:256], v2, v3], axis=0)
        e, m = _edge_mlps(gac, gb, e, wedge, _bias_pack([b2, b3, cb2, cb3]), e_num)

        s2 = _scatter_fn(n, e_pad, k)(m, col3s)

        wn2 = jnp.concatenate([u1[0:128], u1[128:256], u2, u3], axis=0)
        h = _node_update(h, s2[:n], s2[n_pad:n_pad + n], c0, c1, wn2,
                         _bias_pack([d1, d2, d3]))

    return h


# speedup vs baseline: 1.1254x; 1.1083x over previous
import functools

import jax
import jax.numpy as jnp
from jax import lax
from jax.experimental import pallas as pl
from jax.experimental.pallas import tpu as pltpu
from jax.experimental.pallas import tpu_sc as plsc

NC = 2
NS = 16
NW = NC * NS
CHUNK = 128


def _leaky(t):
    return jnp.where(t >= 0, t, 0.01 * t)


def _dot(a, b):
    return jnp.dot(a, b, preferred_element_type=jnp.float32)


def _bias_pack(biases):
    rows = jnp.stack(biases, axis=0)
    return jnp.pad(rows, ((0, 8 - rows.shape[0]), (0, 0)))


def _pack_pair(lo, hi):
    lo16 = lax.bitcast_convert_type(lo.astype(jnp.bfloat16), jnp.uint16)
    hi16 = lax.bitcast_convert_type(hi.astype(jnp.bfloat16), jnp.uint16)
    word = lo16.astype(jnp.uint32) | (hi16.astype(jnp.uint32) << 16)
    return lax.bitcast_convert_type(word, jnp.float32)


def _unpack_pair(w):
    u = lax.bitcast_convert_type(w, jnp.uint32)
    f_lo = lax.bitcast_convert_type(u << 16, jnp.float32)
    f_hi = lax.bitcast_convert_type(u & jnp.uint32(0xFFFF0000), jnp.float32)
    return f_lo, f_hi


def _unpack_cols(w):
    f_lo, f_hi = _unpack_pair(w)
    return jnp.concatenate([f_lo, f_hi], axis=1)


@functools.lru_cache(maxsize=None)
def _gather_fn(n, f, e_pad, k, dtype=jnp.float32):
    ew = e_pad // NW
    mesh = plsc.VectorSubcoreMesh(core_axis_name="c", subcore_axis_name="s")

    @functools.partial(
        pl.kernel,
        mesh=mesh,
        out_type=jax.ShapeDtypeStruct((e_pad, f), dtype),
        scratch_types=[
            pltpu.VMEM((k, CHUNK), jnp.int32),
            pltpu.VMEM((CHUNK, f), dtype),
            pltpu.SemaphoreType.DMA,
        ],
    )
    def gather(table_hbm, idx_hbm, out_hbm, idx_v, rows_v, gsem):
        wid = lax.axis_index("s") * NC + lax.axis_index("c")
        base = wid * ew
        pltpu.sync_copy(idx_hbm.at[wid], idx_v)

        @pl.loop(0, k)
        def _(j):
            pltpu.async_copy(table_hbm.at[idx_v.at[j]], rows_v, gsem).wait()
            pltpu.sync_copy(rows_v, out_hbm.at[pl.ds(base + j * CHUNK, CHUNK)])

    return gather


@functools.lru_cache(maxsize=None)
def _scatter_fn(n, e_pad, k):
    ew = e_pad // NW
    zr = -(-(-(-n // NS)) // 8) * 8
    n_pad = NS * zr
    zfull, zrem = zr // CHUNK, zr % CHUNK
    mesh = plsc.VectorSubcoreMesh(core_axis_name="c", subcore_axis_name="s")

    def _zero_acc(rows_v, acc_sh, sid):
        zero = jnp.zeros((16,), jnp.float32)

        @pl.loop(0, CHUNK)
        def _(r):
            for c8 in range(8):
                rows_v[r, pl.ds(c8 * 16, 16)] = zero

        zb = sid * zr
        for t in range(zfull):
            pltpu.sync_copy(rows_v, acc_sh.at[pl.ds(zb + t * CHUNK, CHUNK)])
        if zrem:
            pltpu.sync_copy(
                rows_v.at[pl.ds(0, zrem)],
                acc_sh.at[pl.ds(zb + zfull * CHUNK, zrem)],
            )

    def _write_acc(rows_v, acc_sh, out_hbm, cid, sid):
        zb = sid * zr
        ob = cid * n_pad + zb
        for t in range(zfull):
            pltpu.sync_copy(acc_sh.at[pl.ds(zb + t * CHUNK, CHUNK)], rows_v)
            pltpu.sync_copy(rows_v, out_hbm.at[pl.ds(ob + t * CHUNK, CHUNK)])
        if zrem:
            pltpu.sync_copy(
                acc_sh.at[pl.ds(zb + zfull * CHUNK, zrem)],
                rows_v.at[pl.ds(0, zrem)],
            )
            pltpu.sync_copy(
                rows_v.at[pl.ds(0, zrem)],
                out_hbm.at[pl.ds(ob + zfull * CHUNK, zrem)],
            )

    @functools.partial(
        pl.kernel,
        mesh=mesh,
        out_type=jax.ShapeDtypeStruct((NC * n_pad, 128), jnp.float32),
        scratch_types=[
            pltpu.VMEM((k, CHUNK), jnp.int32),
            pltpu.VMEM((2, CHUNK, 128), jnp.float32),
            pltpu.VMEM_SHARED((n_pad, 128), jnp.float32),
            pltpu.SemaphoreType.DMA,
            pltpu.SemaphoreType.DMA,
            pltpu.SemaphoreType.DMA,
            pltpu.SemaphoreType.DMA,
        ],
    )
    def scatter(vals_hbm, idx_hbm, out_hbm, idx_v, rows_v, acc_sh,
                ls0, ls1, ss0, ss1):
        cid = lax.axis_index("c")
        sid = lax.axis_index("s")
        wid = sid * NC + cid
        base = wid * ew

        _zero_acc(rows_v.at[0], acc_sh, sid)
        plsc.subcore_barrier()

        pltpu.sync_copy(idx_hbm.at[wid], idx_v)

        @pl.loop(0, k - k % 2, step=2)
        def _(j0):
            for b in range(2):
                j = j0 + b
                lsem = ls0 if b == 0 else ls1
                ssem = ss0 if b == 0 else ss1
                buf = rows_v.at[b]
                src = vals_hbm.at[pl.ds(base + j * CHUNK, CHUNK)]

                @pl.when(j >= 2)
                def _():
                    pltpu.make_async_copy(buf, acc_sh.at[idx_v.at[j]], ssem).wait()

                pltpu.async_copy(src, buf, lsem)
                pltpu.make_async_copy(src, buf, lsem).wait()
                pltpu.async_copy(buf, acc_sh.at[idx_v.at[j]], ssem, add=True)

        for b in range(2):
            ssem = ss0 if b == 0 else ss1
            pltpu.make_async_copy(
                rows_v.at[b], acc_sh.at[idx_v.at[b]], ssem
            ).wait()

        if k % 2:
            pltpu.sync_copy(
                vals_hbm.at[pl.ds(base + (k - 1) * CHUNK, CHUNK)], rows_v.at[0]
            )
            pltpu.sync_copy(rows_v.at[0], acc_sh.at[idx_v.at[k - 1]], add=True)

        plsc.subcore_barrier()
        _write_acc(rows_v.at[0], acc_sh, out_hbm, cid, sid)

    return scatter


@functools.lru_cache(maxsize=None)
def _counts_fn(n, e_pad, k, e_num):
    ew = e_pad // NW
    zr = -(-(-(-n // NS)) // 8) * 8
    n_pad = NS * zr
    zfull, zrem = zr // CHUNK, zr % CHUNK
    mesh = plsc.VectorSubcoreMesh(core_axis_name="c", subcore_axis_name="s")

    @functools.partial(
        pl.kernel,
        mesh=mesh,
        out_type=jax.ShapeDtypeStruct((NC * n_pad, 128), jnp.float32),
        scratch_types=[
            pltpu.VMEM((k, CHUNK), jnp.int32),
            pltpu.VMEM((2, CHUNK, 128), jnp.float32),
            pltpu.VMEM_SHARED((n_pad, 128), jnp.float32),
            pltpu.SemaphoreType.DMA,
        ],
    )
    def counts(idx_hbm, out_hbm, idx_v, rows_v, acc_sh, sem):
        cid = lax.axis_index("c")
        sid = lax.axis_index("s")
        wid = sid * NC + cid
        base = wid * ew
        n_real = jnp.clip(e_num - base, 0, ew)
        kf = n_real // CHUNK
        prem = n_real % CHUNK

        zero = jnp.zeros((16,), jnp.float32)

        @pl.loop(0, CHUNK)
        def _(r):
            for c8 in range(8):
                rows_v[0, r, pl.ds(c8 * 16, 16)] = zero
                rows_v[1, r, pl.ds(c8 * 16, 16)] = jnp.where(
                    r < prem, 1.0, 0.0
                ) * jnp.ones((16,), jnp.float32)

        zb = sid * zr
        for t in range(zfull):
            pltpu.sync_copy(rows_v.at[0], acc_sh.at[pl.ds(zb + t * CHUNK, CHUNK)])
        if zrem:
            pltpu.sync_copy(
                rows_v.at[0, pl.ds(0, zrem)],
                acc_sh.at[pl.ds(zb + zfull * CHUNK, zrem)],
            )
        plsc.subcore_barrier()

        pltpu.sync_copy(idx_hbm.at[wid], idx_v)

        @pl.loop(0, CHUNK)
        def _(r):
            for c8 in range(8):
                rows_v[0, r, pl.ds(c8 * 16, 16)] = jnp.ones((16,), jnp.float32)

        @pl.loop(0, kf)
        def _(j):
            pltpu.sync_copy(rows_v.at[0], acc_sh.at[idx_v.at[j]], add=True)

        @pl.when(prem > 0)
        def _():
            pltpu.sync_copy(rows_v.at[1], acc_sh.at[idx_v.at[kf]], add=True)

        plsc.subcore_barrier()

        ob = cid * n_pad + zb
        for t in range(zfull):
            pltpu.sync_copy(acc_sh.at[pl.ds(zb + t * CHUNK, CHUNK)], rows_v.at[0])
            pltpu.sync_copy(rows_v.at[0], out_hbm.at[pl.ds(ob + t * CHUNK, CHUNK)])
        if zrem:
            pltpu.sync_copy(
                acc_sh.at[pl.ds(zb + zfull * CHUNK, zrem)],
                rows_v.at[0, pl.ds(0, zrem)],
            )
            pltpu.sync_copy(
                rows_v.at[0, pl.ds(0, zrem)],
                out_hbm.at[pl.ds(ob + zfull * CHUNK, zrem)],
            )

    return counts


def _node_pre(h, wpack, bpack):
    n = h.shape[0]
    bn = 2000

    def body(h_ref, w_ref, b_ref, hac_ref, hb_ref):
        hh = h_ref[...]
        ha = _dot(hh, w_ref[0:128]) + b_ref[0:1, :]
        hc = _dot(hh, w_ref[256:384]) + b_ref[1:2, :]
        hb = _dot(hh, w_ref[128:256])
        hac_ref[:, 0:64] = _pack_pair(ha[:, 0:64], ha[:, 64:128])
        hac_ref[:, 64:128] = _pack_pair(hc[:, 0:64], hc[:, 64:128])
        hb_ref[...] = hb

    return pl.pallas_call(
        body,
        grid=(n // bn,),
        in_specs=[
            pl.BlockSpec((bn, 128), lambda i: (i, 0)),
            pl.BlockSpec((384, 128), lambda i: (0, 0)),
            pl.BlockSpec((8, 128), lambda i: (0, 0)),
        ],
        out_specs=[
            pl.BlockSpec((bn, 128), lambda i: (i, 0)),
            pl.BlockSpec((bn, 128), lambda i: (i, 0)),
        ],
        out_shape=[
            jax.ShapeDtypeStruct((n, 128), jnp.float32),
            jax.ShapeDtypeStruct((n, 128), jnp.float32),
        ],
        compiler_params=pltpu.CompilerParams(dimension_semantics=("parallel",)),
    )(h, wpack, bpack)


def _edge_mlps(gac, gb, e, wpack, bpack, e_real):
    e_pad = e.shape[0]
    be = 1024

    def body(gac_ref, gb_ref, e_ref, w_ref, b_ref, enew_ref, m_ref):
        i = pl.program_id(0)
        ga = _unpack_cols(gac_ref[:, 0:64])
        gc = _unpack_cols(gac_ref[:, 64:128])
        u = _leaky(ga + gb_ref[...] + _dot(e_ref[...], w_ref[0:128]))
        u = _leaky(_dot(u, w_ref[128:256]) + b_ref[0:1, :])
        en = _dot(u, w_ref[256:384]) + b_ref[1:2, :]
        enew_ref[...] = en
        v = _leaky(gc + _dot(en, w_ref[384:512]))
        v = _leaky(_dot(v, w_ref[512:640]) + b_ref[2:3, :])
        m = _dot(v, w_ref[640:768]) + b_ref[3:4, :]
        rowid = i * be + lax.broadcasted_iota(jnp.int32, (be, 1), 0)
        m_ref[...] = jnp.where(rowid < e_real, m, 0.0)

    blk = pl.BlockSpec((be, 128), lambda i: (i, 0))
    osh = jax.ShapeDtypeStruct((e_pad, 128), jnp.float32)
    return pl.pallas_call(
        body,
        grid=(e_pad // be,),
        in_specs=[
            blk, blk, blk,
            pl.BlockSpec((768, 128), lambda i: (0, 0)),
            pl.BlockSpec((8, 128), lambda i: (0, 0)),
        ],
        out_specs=[blk, blk],
        out_shape=[osh, osh],
        compiler_params=pltpu.CompilerParams(dimension_semantics=("parallel",)),
    )(gac, gb, e, wpack, bpack)


def _node_update(h, s0, s1, c0, c1, wpack, bpack):
    n = h.shape[0]
    bn = 2000

    def body(h_ref, s0_ref, s1_ref, c0_ref, c1_ref, w_ref, b_ref, out_ref):
        cnt = jnp.maximum(c0_ref[...] + c1_ref[...], 1.0)
        agg = (s0_ref[...] + s1_ref[...]) / cnt
        t = _leaky(
            _dot(h_ref[...], w_ref[0:128]) + _dot(agg, w_ref[128:256]) + b_ref[0:1, :]
        )
        t = _leaky(_dot(t, w_ref[256:384]) + b_ref[1:2, :])
        out_ref[...] = _dot(t, w_ref[384:512]) + b_ref[2:3, :]

    blk = pl.BlockSpec((bn, 128), lambda i: (i, 0))
    return pl.pallas_call(
        body,
        grid=(n // bn,),
        in_specs=[
            blk, blk, blk, blk, blk,
            pl.BlockSpec((512, 128), lambda i: (0, 0)),
            pl.BlockSpec((8, 128), lambda i: (0, 0)),
        ],
        out_specs=blk,
        out_shape=jax.ShapeDtypeStruct((n, 128), jnp.float32),
        compiler_params=pltpu.CompilerParams(dimension_semantics=("parallel",)),
    )(h, s0, s1, c0, c1, wpack, bpack)


def kernel(x, edge_index, edge_attr, params):
    n, d = x.shape
    e_num = edge_attr.shape[0]
    k = -(-e_num // (NW * CHUNK))
    e_pad = NW * CHUNK * k
    pad = e_pad - e_num

    row = edge_index[0].astype(jnp.int32)
    col = edge_index[1].astype(jnp.int32)
    row3 = jnp.pad(row, (0, pad)).reshape(NW, k, CHUNK)
    col3 = jnp.pad(col, (0, pad)).reshape(NW, k, CHUNK)
    e = jnp.pad(edge_attr, ((0, pad), (0, 0)))

    n_pad = NS * (-(-(-(-n // NS)) // 8) * 8)
    spread = max(n_pad - n, 1)
    pad_idx = n_pad - 1 - (jnp.arange(pad, dtype=jnp.int32) % spread)
    col3s = jnp.concatenate([col, pad_idx]).reshape(NW, k, CHUNK)

    cnt2 = _counts_fn(n, e_pad, k, e_num)(col3s)
    c0, c1 = cnt2[:n], cnt2[n_pad:n_pad + n]

    h = x
    for lp in params:
        (w1, b1), (w2, b2), (w3, b3) = lp["edge"]
        (v1, cb1), (v2, cb2), (v3, cb3) = lp["node1"]
        (u1, d1), (u2, d2), (u3, d3) = lp["node2"]

        wpre = jnp.concatenate([w1[0:128], w1[128:256], v1[0:128]], axis=0)
        hac, hb = _node_pre(h, wpre, _bias_pack([b1, cb1]))

        gac = _gather_fn(n, 128, e_pad, k)(hac, row3)
        gb = _gather_fn(n, 128, e_pad, k)(hb, col3)

        wedge = jnp.concatenate([w1[256:384], w2, w3, v1[128:256], v2, v3], axis=0)
        e, m = _edge_mlps(gac, gb, e, wedge, _bias_pack([b2, b3, cb2, cb3]), e_num)

        s2 = _scatter_fn(n, e_pad, k)(m, col3s)

        wn2 = jnp.concatenate([u1[0:128], u1[128:256], u2, u3], axis=0)
        h = _node_update(h, s2[:n], s2[n_pad:n_pad + n], c0, c1, wn2,
                         _bias_pack([d1, d2, d3]))

    return h


# speedup vs baseline: 1.1776x; 1.0464x over previous
import functools

import jax
import jax.numpy as jnp
from jax import lax
from jax.experimental import pallas as pl
from jax.experimental.pallas import tpu as pltpu
from jax.experimental.pallas import tpu_sc as plsc

NC = 2
NS = 16
NW = NC * NS
CHUNK = 128


def _leaky(t):
    return jnp.where(t >= 0, t, 0.01 * t)


def _dot(a, b):
    return jnp.dot(a, b, preferred_element_type=jnp.float32)


def _bias_pack(biases):
    rows = jnp.stack(biases, axis=0)
    return jnp.pad(rows, ((0, 8 - rows.shape[0]), (0, 0)))


def _pack_pair(lo, hi):
    lo16 = lax.bitcast_convert_type(lo.astype(jnp.bfloat16), jnp.uint16)
    hi16 = lax.bitcast_convert_type(hi.astype(jnp.bfloat16), jnp.uint16)
    word = lo16.astype(jnp.uint32) | (hi16.astype(jnp.uint32) << 16)
    return lax.bitcast_convert_type(word, jnp.float32)


def _unpack_pair(w):
    u = lax.bitcast_convert_type(w, jnp.uint32)
    f_lo = lax.bitcast_convert_type(u << 16, jnp.float32)
    f_hi = lax.bitcast_convert_type(u & jnp.uint32(0xFFFF0000), jnp.float32)
    return f_lo, f_hi


def _unpack_cols(w):
    f_lo, f_hi = _unpack_pair(w)
    return jnp.concatenate([f_lo, f_hi], axis=1)


@functools.lru_cache(maxsize=None)
def _gather_fn(n, f, e_pad, k, dtype=jnp.float32):
    ew = e_pad // NW
    mesh = plsc.VectorSubcoreMesh(core_axis_name="c", subcore_axis_name="s")

    @functools.partial(
        pl.kernel,
        mesh=mesh,
        out_type=jax.ShapeDtypeStruct((e_pad, f), dtype),
        scratch_types=[
            pltpu.VMEM((k, CHUNK), jnp.int32),
            pltpu.VMEM((2, CHUNK, f), dtype),
            pltpu.SemaphoreType.DMA,
            pltpu.SemaphoreType.DMA,
            pltpu.SemaphoreType.DMA,
        ],
    )
    def gather(table_hbm, idx_hbm, out_hbm, idx_v, rows_v, gsem, ws0, ws1):
        wid = lax.axis_index("s") * NC + lax.axis_index("c")
        base = wid * ew
        pltpu.sync_copy(idx_hbm.at[wid], idx_v)

        @pl.loop(0, k - k % 2, step=2)
        def _(j0):
            for b in range(2):
                j = j0 + b
                wsem = ws0 if b == 0 else ws1
                buf = rows_v.at[b]
                dst = out_hbm.at[pl.ds(base + j * CHUNK, CHUNK)]

                @pl.when(j >= 2)
                def _():
                    pltpu.make_async_copy(buf, dst, wsem).wait()

                pltpu.async_copy(table_hbm.at[idx_v.at[j]], buf, gsem)
                pltpu.make_async_copy(table_hbm.at[idx_v.at[j]], buf, gsem).wait()
                pltpu.async_copy(buf, dst, wsem)

        for b in range(2):
            wsem = ws0 if b == 0 else ws1
            pltpu.make_async_copy(
                rows_v.at[b],
                out_hbm.at[pl.ds(base + b * CHUNK, CHUNK)],
                wsem,
            ).wait()

        if k % 2:
            j = k - 1
            pltpu.async_copy(table_hbm.at[idx_v.at[j]], rows_v.at[0], gsem).wait()
            pltpu.sync_copy(rows_v.at[0], out_hbm.at[pl.ds(base + j * CHUNK, CHUNK)])

    return gather


@functools.lru_cache(maxsize=None)
def _scatter_fn(n, e_pad, k):
    ew = e_pad // NW
    zr = -(-(-(-n // NS)) // 8) * 8
    n_pad = NS * zr
    zfull, zrem = zr // CHUNK, zr % CHUNK
    mesh = plsc.VectorSubcoreMesh(core_axis_name="c", subcore_axis_name="s")

    def _zero_acc(rows_v, acc_sh, sid):
        zero = jnp.zeros((16,), jnp.float32)

        @pl.loop(0, CHUNK)
        def _(r):
            for c8 in range(8):
                rows_v[r, pl.ds(c8 * 16, 16)] = zero

        zb = sid * zr
        for t in range(zfull):
            pltpu.sync_copy(rows_v, acc_sh.at[pl.ds(zb + t * CHUNK, CHUNK)])
        if zrem:
            pltpu.sync_copy(
                rows_v.at[pl.ds(0, zrem)],
                acc_sh.at[pl.ds(zb + zfull * CHUNK, zrem)],
            )

    def _write_acc(rows_v, acc_sh, out_hbm, cid, sid):
        zb = sid * zr
        ob = cid * n_pad + zb
        for t in range(zfull):
            pltpu.sync_copy(acc_sh.at[pl.ds(zb + t * CHUNK, CHUNK)], rows_v)
            pltpu.sync_copy(rows_v, out_hbm.at[pl.ds(ob + t * CHUNK, CHUNK)])
        if zrem:
            pltpu.sync_copy(
                acc_sh.at[pl.ds(zb + zfull * CHUNK, zrem)],
                rows_v.at[pl.ds(0, zrem)],
            )
            pltpu.sync_copy(
                rows_v.at[pl.ds(0, zrem)],
                out_hbm.at[pl.ds(ob + zfull * CHUNK, zrem)],
            )

    @functools.partial(
        pl.kernel,
        mesh=mesh,
        out_type=jax.ShapeDtypeStruct((NC * n_pad, 128), jnp.float32),
        scratch_types=[
            pltpu.VMEM((k, CHUNK), jnp.int32),
            pltpu.VMEM((2, CHUNK, 128), jnp.float32),
            pltpu.VMEM_SHARED((n_pad, 128), jnp.float32),
            pltpu.SemaphoreType.DMA,
            pltpu.SemaphoreType.DMA,
            pltpu.SemaphoreType.DMA,
            pltpu.SemaphoreType.DMA,
        ],
    )
    def scatter(vals_hbm, idx_hbm, out_hbm, idx_v, rows_v, acc_sh,
                ls0, ls1, ss0, ss1):
        cid = lax.axis_index("c")
        sid = lax.axis_index("s")
        wid = sid * NC + cid
        base = wid * ew

        _zero_acc(rows_v.at[0], acc_sh, sid)
        plsc.subcore_barrier()

        pltpu.sync_copy(idx_hbm.at[wid], idx_v)

        @pl.loop(0, k - k % 2, step=2)
        def _(j0):
            for b in range(2):
                j = j0 + b
                lsem = ls0 if b == 0 else ls1
                ssem = ss0 if b == 0 else ss1
                buf = rows_v.at[b]
                src = vals_hbm.at[pl.ds(base + j * CHUNK, CHUNK)]

                @pl.when(j >= 2)
                def _():
                    pltpu.make_async_copy(buf, acc_sh.at[idx_v.at[j]], ssem).wait()

                pltpu.async_copy(src, buf, lsem)
                pltpu.make_async_copy(src, buf, lsem).wait()
                pltpu.async_copy(buf, acc_sh.at[idx_v.at[j]], ssem, add=True)

        for b in range(2):
            ssem = ss0 if b == 0 else ss1
            pltpu.make_async_copy(
                rows_v.at[b], acc_sh.at[idx_v.at[b]], ssem
            ).wait()

        if k % 2:
            pltpu.sync_copy(
                vals_hbm.at[pl.ds(base + (k - 1) * CHUNK, CHUNK)], rows_v.at[0]
            )
            pltpu.sync_copy(rows_v.at[0], acc_sh.at[idx_v.at[k - 1]], add=True)

        plsc.subcore_barrier()
        _write_acc(rows_v.at[0], acc_sh, out_hbm, cid, sid)

    return scatter


@functools.lru_cache(maxsize=None)
def _counts_fn(n, e_pad, k, e_num):
    ew = e_pad // NW
    zr = -(-(-(-n // NS)) // 8) * 8
    n_pad = NS * zr
    zfull, zrem = zr // CHUNK, zr % CHUNK
    mesh = plsc.VectorSubcoreMesh(core_axis_name="c", subcore_axis_name="s")

    @functools.partial(
        pl.kernel,
        mesh=mesh,
        out_type=jax.ShapeDtypeStruct((NC * n_pad, 128), jnp.float32),
        scratch_types=[
            pltpu.VMEM((k, CHUNK), jnp.int32),
            pltpu.VMEM((2, CHUNK, 128), jnp.float32),
            pltpu.VMEM_SHARED((n_pad, 128), jnp.float32),
            pltpu.SemaphoreType.DMA,
        ],
    )
    def counts(idx_hbm, out_hbm, idx_v, rows_v, acc_sh, sem):
        cid = lax.axis_index("c")
        sid = lax.axis_index("s")
        wid = sid * NC + cid
        base = wid * ew
        n_real = jnp.clip(e_num - base, 0, ew)
        kf = n_real // CHUNK
        prem = n_real % CHUNK

        zero = jnp.zeros((16,), jnp.float32)

        @pl.loop(0, CHUNK)
        def _(r):
            for c8 in range(8):
                rows_v[0, r, pl.ds(c8 * 16, 16)] = zero
                rows_v[1, r, pl.ds(c8 * 16, 16)] = jnp.where(
                    r < prem, 1.0, 0.0
                ) * jnp.ones((16,), jnp.float32)

        zb = sid * zr
        for t in range(zfull):
            pltpu.sync_copy(rows_v.at[0], acc_sh.at[pl.ds(zb + t * CHUNK, CHUNK)])
        if zrem:
            pltpu.sync_copy(
                rows_v.at[0, pl.ds(0, zrem)],
                acc_sh.at[pl.ds(zb + zfull * CHUNK, zrem)],
            )
        plsc.subcore_barrier()

        pltpu.sync_copy(idx_hbm.at[wid], idx_v)

        @pl.loop(0, CHUNK)
        def _(r):
            for c8 in range(8):
                rows_v[0, r, pl.ds(c8 * 16, 16)] = jnp.ones((16,), jnp.float32)

        @pl.loop(0, kf)
        def _(j):
            pltpu.sync_copy(rows_v.at[0], acc_sh.at[idx_v.at[j]], add=True)

        @pl.when(prem > 0)
        def _():
            pltpu.sync_copy(rows_v.at[1], acc_sh.at[idx_v.at[kf]], add=True)

        plsc.subcore_barrier()

        ob = cid * n_pad + zb
        for t in range(zfull):
            pltpu.sync_copy(acc_sh.at[pl.ds(zb + t * CHUNK, CHUNK)], rows_v.at[0])
            pltpu.sync_copy(rows_v.at[0], out_hbm.at[pl.ds(ob + t * CHUNK, CHUNK)])
        if zrem:
            pltpu.sync_copy(
                acc_sh.at[pl.ds(zb + zfull * CHUNK, zrem)],
                rows_v.at[0, pl.ds(0, zrem)],
            )
            pltpu.sync_copy(
                rows_v.at[0, pl.ds(0, zrem)],
                out_hbm.at[pl.ds(ob + zfull * CHUNK, zrem)],
            )

    return counts


def _node_pre(h, wpack, bpack):
    n = h.shape[0]
    bn = 2000

    def body(h_ref, w_ref, b_ref, hac_ref, hb_ref):
        hh = h_ref[...]
        ha = _dot(hh, w_ref[0:128]) + b_ref[0:1, :]
        hc = _dot(hh, w_ref[256:384]) + b_ref[1:2, :]
        hb = _dot(hh, w_ref[128:256])
        hac_ref[:, 0:64] = _pack_pair(ha[:, 0:64], ha[:, 64:128])
        hac_ref[:, 64:128] = _pack_pair(hc[:, 0:64], hc[:, 64:128])
        hb_ref[...] = hb

    return pl.pallas_call(
        body,
        grid=(n // bn,),
        in_specs=[
            pl.BlockSpec((bn, 128), lambda i: (i, 0)),
            pl.BlockSpec((384, 128), lambda i: (0, 0)),
            pl.BlockSpec((8, 128), lambda i: (0, 0)),
        ],
        out_specs=[
            pl.BlockSpec((bn, 128), lambda i: (i, 0)),
            pl.BlockSpec((bn, 128), lambda i: (i, 0)),
        ],
        out_shape=[
            jax.ShapeDtypeStruct((n, 128), jnp.float32),
            jax.ShapeDtypeStruct((n, 128), jnp.float32),
        ],
        compiler_params=pltpu.CompilerParams(dimension_semantics=("parallel",)),
    )(h, wpack, bpack)


def _edge_mlps(gac, gb, e, wpack, bpack, e_real):
    e_pad = e.shape[0]
    be = 1024

    def body(gac_ref, gb_ref, e_ref, w_ref, b_ref, enew_ref, m_ref):
        i = pl.program_id(0)
        ga = _unpack_cols(gac_ref[:, 0:64])
        gc = _unpack_cols(gac_ref[:, 64:128])
        u = _leaky(ga + gb_ref[...] + _dot(e_ref[...], w_ref[0:128]))
        u = _leaky(_dot(u, w_ref[128:256]) + b_ref[0:1, :])
        en = _dot(u, w_ref[256:384]) + b_ref[1:2, :]
        enew_ref[...] = en
        v = _leaky(gc + _dot(en, w_ref[384:512]))
        v = _leaky(_dot(v, w_ref[512:640]) + b_ref[2:3, :])
        m = _dot(v, w_ref[640:768]) + b_ref[3:4, :]
        rowid = i * be + lax.broadcasted_iota(jnp.int32, (be, 1), 0)
        m_ref[...] = jnp.where(rowid < e_real, m, 0.0)

    blk = pl.BlockSpec((be, 128), lambda i: (i, 0))
    osh = jax.ShapeDtypeStruct((e_pad, 128), jnp.float32)
    return pl.pallas_call(
        body,
        grid=(e_pad // be,),
        in_specs=[
            blk, blk, blk,
            pl.BlockSpec((768, 128), lambda i: (0, 0)),
            pl.BlockSpec((8, 128), lambda i: (0, 0)),
        ],
        out_specs=[blk, blk],
        out_shape=[osh, osh],
        compiler_params=pltpu.CompilerParams(dimension_semantics=("parallel",)),
    )(gac, gb, e, wpack, bpack)


def _node_update(h, s0, s1, c0, c1, wpack, bpack):
    n = h.shape[0]
    bn = 2000

    def body(h_ref, s0_ref, s1_ref, c0_ref, c1_ref, w_ref, b_ref, out_ref):
        cnt = jnp.maximum(c0_ref[...] + c1_ref[...], 1.0)
        agg = (s0_ref[...] + s1_ref[...]) / cnt
        t = _leaky(
            _dot(h_ref[...], w_ref[0:128]) + _dot(agg, w_ref[128:256]) + b_ref[0:1, :]
        )
        t = _leaky(_dot(t, w_ref[256:384]) + b_ref[1:2, :])
        out_ref[...] = _dot(t, w_ref[384:512]) + b_ref[2:3, :]

    blk = pl.BlockSpec((bn, 128), lambda i: (i, 0))
    return pl.pallas_call(
        body,
        grid=(n // bn,),
        in_specs=[
            blk, blk, blk, blk, blk,
            pl.BlockSpec((512, 128), lambda i: (0, 0)),
            pl.BlockSpec((8, 128), lambda i: (0, 0)),
        ],
        out_specs=blk,
        out_shape=jax.ShapeDtypeStruct((n, 128), jnp.float32),
        compiler_params=pltpu.CompilerParams(dimension_semantics=("parallel",)),
    )(h, s0, s1, c0, c1, wpack, bpack)


def kernel(x, edge_index, edge_attr, params):
    n, d = x.shape
    e_num = edge_attr.shape[0]
    k = -(-e_num // (NW * CHUNK))
    e_pad = NW * CHUNK * k
    pad = e_pad - e_num

    row = edge_index[0].astype(jnp.int32)
    col = edge_index[1].astype(jnp.int32)
    row3 = jnp.pad(row, (0, pad)).reshape(NW, k, CHUNK)
    col3 = jnp.pad(col, (0, pad)).reshape(NW, k, CHUNK)
    e = jnp.pad(edge_attr, ((0, pad), (0, 0)))

    n_pad = NS * (-(-(-(-n // NS)) // 8) * 8)
    spread = max(n_pad - n, 1)
    pad_idx = n_pad - 1 - (jnp.arange(pad, dtype=jnp.int32) % spread)
    col3s = jnp.concatenate([col, pad_idx]).reshape(NW, k, CHUNK)

    cnt2 = _counts_fn(n, e_pad, k, e_num)(col3s)
    c0, c1 = cnt2[:n], cnt2[n_pad:n_pad + n]

    h = x
    for lp in params:
        (w1, b1), (w2, b2), (w3, b3) = lp["edge"]
        (v1, cb1), (v2, cb2), (v3, cb3) = lp["node1"]
        (u1, d1), (u2, d2), (u3, d3) = lp["node2"]

        wpre = jnp.concatenate([w1[0:128], w1[128:256], v1[0:128]], axis=0)
        hac, hb = _node_pre(h, wpre, _bias_pack([b1, cb1]))

        gac = _gather_fn(n, 128, e_pad, k)(hac, row3)
        gb = _gather_fn(n, 128, e_pad, k)(hb, col3)

        wedge = jnp.concatenate([w1[256:384], w2, w3, v1[128:256], v2, v3], axis=0)
        e, m = _edge_mlps(gac, gb, e, wedge, _bias_pack([b2, b3, cb2, cb3]), e_num)

        s2 = _scatter_fn(n, e_pad, k)(m, col3s)

        wn2 = jnp.concatenate([u1[0:128], u1[128:256], u2, u3], axis=0)
        h = _node_update(h, s2[:n], s2[n_pad:n_pad + n], c0, c1, wn2,
                         _bias_pack([d1, d2, d3]))

    return h


# speedup vs baseline: 1.2963x; 1.1008x over previous
import functools

import jax
import jax.numpy as jnp
from jax import lax
from jax.experimental import pallas as pl
from jax.experimental.pallas import tpu as pltpu
from jax.experimental.pallas import tpu_sc as plsc

NC = 2
NS = 16
NW = NC * NS
CHUNK = 128


def _leaky(t):
    return jnp.where(t >= 0, t, 0.01 * t)


def _dot(a, b):
    return jnp.dot(a, b, preferred_element_type=jnp.float32)


def _bias_pack(biases):
    rows = jnp.stack(biases, axis=0)
    return jnp.pad(rows, ((0, 8 - rows.shape[0]), (0, 0)))


def _pack_pair(lo, hi):
    lo16 = lax.bitcast_convert_type(lo.astype(jnp.bfloat16), jnp.uint16)
    hi16 = lax.bitcast_convert_type(hi.astype(jnp.bfloat16), jnp.uint16)
    word = lo16.astype(jnp.uint32) | (hi16.astype(jnp.uint32) << 16)
    return lax.bitcast_convert_type(word, jnp.float32)


def _unpack_pair(w):
    u = lax.bitcast_convert_type(w, jnp.uint32)
    f_lo = lax.bitcast_convert_type(u << 16, jnp.float32)
    f_hi = lax.bitcast_convert_type(u & jnp.uint32(0xFFFF0000), jnp.float32)
    return f_lo, f_hi


def _unpack_cols(w):
    f_lo, f_hi = _unpack_pair(w)
    return jnp.concatenate([f_lo, f_hi], axis=1)


@functools.lru_cache(maxsize=None)
def _gather_fn(n, f, e_pad, k, dtype=jnp.float32):
    ew = e_pad // NW
    mesh = plsc.VectorSubcoreMesh(core_axis_name="c", subcore_axis_name="s")

    @functools.partial(
        pl.kernel,
        mesh=mesh,
        out_type=jax.ShapeDtypeStruct((e_pad, f), dtype),
        scratch_types=[
            pltpu.VMEM((k, CHUNK), jnp.int32),
            pltpu.VMEM((2, CHUNK, f), dtype),
            pltpu.SemaphoreType.DMA,
            pltpu.SemaphoreType.DMA,
            pltpu.SemaphoreType.DMA,
        ],
    )
    def gather(table_hbm, idx_hbm, out_hbm, idx_v, rows_v, gsem, ws0, ws1):
        wid = lax.axis_index("s") * NC + lax.axis_index("c")
        base = wid * ew
        pltpu.sync_copy(idx_hbm.at[wid], idx_v)

        @pl.loop(0, k - k % 2, step=2)
        def _(j0):
            for b in range(2):
                j = j0 + b
                wsem = ws0 if b == 0 else ws1
                buf = rows_v.at[b]
                dst = out_hbm.at[pl.ds(base + j * CHUNK, CHUNK)]

                @pl.when(j >= 2)
                def _():
                    pltpu.make_async_copy(buf, dst, wsem).wait()

                pltpu.async_copy(table_hbm.at[idx_v.at[j]], buf, gsem)
                pltpu.make_async_copy(table_hbm.at[idx_v.at[j]], buf, gsem).wait()
                pltpu.async_copy(buf, dst, wsem)

        for b in range(2):
            wsem = ws0 if b == 0 else ws1
            pltpu.make_async_copy(
                rows_v.at[b],
                out_hbm.at[pl.ds(base + b * CHUNK, CHUNK)],
                wsem,
            ).wait()

        if k % 2:
            j = k - 1
            pltpu.async_copy(table_hbm.at[idx_v.at[j]], rows_v.at[0], gsem).wait()
            pltpu.sync_copy(rows_v.at[0], out_hbm.at[pl.ds(base + j * CHUNK, CHUNK)])

    return gather


@functools.lru_cache(maxsize=None)
def _scatter_fn(n, e_pad, k):
    ew = e_pad // NW
    zr = -(-(-(-n // NS)) // 8) * 8
    n_pad = NS * zr
    zfull, zrem = zr // CHUNK, zr % CHUNK
    mesh = plsc.VectorSubcoreMesh(core_axis_name="c", subcore_axis_name="s")

    def _zero_acc(rows_v, acc_sh, sid):
        zero = jnp.zeros((16,), jnp.float32)

        @pl.loop(0, CHUNK)
        def _(r):
            for c8 in range(8):
                rows_v[r, pl.ds(c8 * 16, 16)] = zero

        zb = sid * zr
        for t in range(zfull):
            pltpu.sync_copy(rows_v, acc_sh.at[pl.ds(zb + t * CHUNK, CHUNK)])
        if zrem:
            pltpu.sync_copy(
                rows_v.at[pl.ds(0, zrem)],
                acc_sh.at[pl.ds(zb + zfull * CHUNK, zrem)],
            )

    def _write_acc(rows_v, acc_sh, out_hbm, cid, sid):
        zb = sid * zr
        ob = cid * n_pad + zb
        for t in range(zfull):
            pltpu.sync_copy(acc_sh.at[pl.ds(zb + t * CHUNK, CHUNK)], rows_v)
            pltpu.sync_copy(rows_v, out_hbm.at[pl.ds(ob + t * CHUNK, CHUNK)])
        if zrem:
            pltpu.sync_copy(
                acc_sh.at[pl.ds(zb + zfull * CHUNK, zrem)],
                rows_v.at[pl.ds(0, zrem)],
            )
            pltpu.sync_copy(
                rows_v.at[pl.ds(0, zrem)],
                out_hbm.at[pl.ds(ob + zfull * CHUNK, zrem)],
            )

    @functools.partial(
        pl.kernel,
        mesh=mesh,
        out_type=jax.ShapeDtypeStruct((NC * n_pad, 128), jnp.float32),
        scratch_types=[
            pltpu.VMEM((k, CHUNK), jnp.int32),
            pltpu.VMEM((2, CHUNK, 128), jnp.float32),
            pltpu.VMEM_SHARED((n_pad, 128), jnp.float32),
            pltpu.SemaphoreType.DMA,
            pltpu.SemaphoreType.DMA,
            pltpu.SemaphoreType.DMA,
            pltpu.SemaphoreType.DMA,
        ],
    )
    def scatter(vals_hbm, idx_hbm, out_hbm, idx_v, rows_v, acc_sh,
                ls0, ls1, ss0, ss1):
        cid = lax.axis_index("c")
        sid = lax.axis_index("s")
        wid = sid * NC + cid
        base = wid * ew

        _zero_acc(rows_v.at[0], acc_sh, sid)
        plsc.subcore_barrier()

        pltpu.sync_copy(idx_hbm.at[wid], idx_v)

        @pl.loop(0, k - k % 2, step=2)
        def _(j0):
            for b in range(2):
                j = j0 + b
                lsem = ls0 if b == 0 else ls1
                ssem = ss0 if b == 0 else ss1
                buf = rows_v.at[b]
                src = vals_hbm.at[pl.ds(base + j * CHUNK, CHUNK)]

                @pl.when(j >= 2)
                def _():
                    pltpu.make_async_copy(buf, acc_sh.at[idx_v.at[j]], ssem).wait()

                pltpu.async_copy(src, buf, lsem)
                pltpu.make_async_copy(src, buf, lsem).wait()
                pltpu.async_copy(buf, acc_sh.at[idx_v.at[j]], ssem, add=True)

        for b in range(2):
            ssem = ss0 if b == 0 else ss1
            pltpu.make_async_copy(
                rows_v.at[b], acc_sh.at[idx_v.at[b]], ssem
            ).wait()

        if k % 2:
            pltpu.sync_copy(
                vals_hbm.at[pl.ds(base + (k - 1) * CHUNK, CHUNK)], rows_v.at[0]
            )
            pltpu.sync_copy(rows_v.at[0], acc_sh.at[idx_v.at[k - 1]], add=True)

        plsc.subcore_barrier()
        _write_acc(rows_v.at[0], acc_sh, out_hbm, cid, sid)

    return scatter


@functools.lru_cache(maxsize=None)
def _counts_fn(n, e_pad, k, e_num):
    ew = e_pad // NW
    zr = -(-(-(-n // NS)) // 8) * 8
    n_pad = NS * zr
    zfull, zrem = zr // CHUNK, zr % CHUNK
    mesh = plsc.VectorSubcoreMesh(core_axis_name="c", subcore_axis_name="s")

    @functools.partial(
        pl.kernel,
        mesh=mesh,
        out_type=jax.ShapeDtypeStruct((NC * n_pad, 128), jnp.float32),
        scratch_types=[
            pltpu.VMEM((k, CHUNK), jnp.int32),
            pltpu.VMEM((2, CHUNK, 128), jnp.float32),
            pltpu.VMEM_SHARED((n_pad, 128), jnp.float32),
            pltpu.SemaphoreType.DMA,
        ],
    )
    def counts(idx_hbm, out_hbm, idx_v, rows_v, acc_sh, sem):
        cid = lax.axis_index("c")
        sid = lax.axis_index("s")
        wid = sid * NC + cid
        base = wid * ew
        n_real = jnp.clip(e_num - base, 0, ew)
        kf = n_real // CHUNK
        prem = n_real % CHUNK

        zero = jnp.zeros((16,), jnp.float32)

        @pl.loop(0, CHUNK)
        def _(r):
            for c8 in range(8):
                rows_v[0, r, pl.ds(c8 * 16, 16)] = zero
                rows_v[1, r, pl.ds(c8 * 16, 16)] = jnp.where(
                    r < prem, 1.0, 0.0
                ) * jnp.ones((16,), jnp.float32)

        zb = sid * zr
        for t in range(zfull):
            pltpu.sync_copy(rows_v.at[0], acc_sh.at[pl.ds(zb + t * CHUNK, CHUNK)])
        if zrem:
            pltpu.sync_copy(
                rows_v.at[0, pl.ds(0, zrem)],
                acc_sh.at[pl.ds(zb + zfull * CHUNK, zrem)],
            )
        plsc.subcore_barrier()

        pltpu.sync_copy(idx_hbm.at[wid], idx_v)

        @pl.loop(0, CHUNK)
        def _(r):
            for c8 in range(8):
                rows_v[0, r, pl.ds(c8 * 16, 16)] = jnp.ones((16,), jnp.float32)

        @pl.loop(0, kf)
        def _(j):
            pltpu.sync_copy(rows_v.at[0], acc_sh.at[idx_v.at[j]], add=True)

        @pl.when(prem > 0)
        def _():
            pltpu.sync_copy(rows_v.at[1], acc_sh.at[idx_v.at[kf]], add=True)

        plsc.subcore_barrier()

        ob = cid * n_pad + zb
        for t in range(zfull):
            pltpu.sync_copy(acc_sh.at[pl.ds(zb + t * CHUNK, CHUNK)], rows_v.at[0])
            pltpu.sync_copy(rows_v.at[0], out_hbm.at[pl.ds(ob + t * CHUNK, CHUNK)])
        if zrem:
            pltpu.sync_copy(
                acc_sh.at[pl.ds(zb + zfull * CHUNK, zrem)],
                rows_v.at[0, pl.ds(0, zrem)],
            )
            pltpu.sync_copy(
                rows_v.at[0, pl.ds(0, zrem)],
                out_hbm.at[pl.ds(ob + zfull * CHUNK, zrem)],
            )

    return counts


def _node_pre(h, wpack, bpack):
    n = h.shape[0]
    bn = 2000

    def body(h_ref, w_ref, b_ref, hac_ref, hb_ref):
        hh = h_ref[...]
        ha = _dot(hh, w_ref[0:128]) + b_ref[0:1, :]
        hc = _dot(hh, w_ref[256:384]) + b_ref[1:2, :]
        hb = _dot(hh, w_ref[128:256])
        hac_ref[:, 0:64] = _pack_pair(ha[:, 0:64], ha[:, 64:128])
        hac_ref[:, 64:128] = _pack_pair(hc[:, 0:64], hc[:, 64:128])
        hb_ref[...] = hb

    return pl.pallas_call(
        body,
        grid=(n // bn,),
        in_specs=[
            pl.BlockSpec((bn, 128), lambda i: (i, 0)),
            pl.BlockSpec((384, 128), lambda i: (0, 0)),
            pl.BlockSpec((8, 128), lambda i: (0, 0)),
        ],
        out_specs=[
            pl.BlockSpec((bn, 128), lambda i: (i, 0)),
            pl.BlockSpec((bn, 128), lambda i: (i, 0)),
        ],
        out_shape=[
            jax.ShapeDtypeStruct((n, 128), jnp.float32),
            jax.ShapeDtypeStruct((n, 128), jnp.float32),
        ],
        compiler_params=pltpu.CompilerParams(dimension_semantics=("parallel",)),
    )(h, wpack, bpack)


def _edge_mlps(gac, gb, e, wpack, bpack, e_real):
    e_pad = e.shape[0]
    be = 2048

    def body(gac_ref, gb_ref, e_ref, w_ref, b_ref, enew_ref, m_ref):
        i = pl.program_id(0)
        ga = _unpack_cols(gac_ref[:, 0:64])
        gc = _unpack_cols(gac_ref[:, 64:128])
        u = _leaky(ga + gb_ref[...] + _dot(e_ref[...], w_ref[0:128]))
        u = _leaky(_dot(u, w_ref[128:256]) + b_ref[0:1, :])
        en = _dot(u, w_ref[256:384]) + b_ref[1:2, :]
        enew_ref[...] = en
        v = _leaky(gc + _dot(en, w_ref[384:512]))
        v = _leaky(_dot(v, w_ref[512:640]) + b_ref[2:3, :])
        m = _dot(v, w_ref[640:768]) + b_ref[3:4, :]
        rowid = i * be + lax.broadcasted_iota(jnp.int32, (be, 1), 0)
        m_ref[...] = jnp.where(rowid < e_real, m, 0.0)

    blk = pl.BlockSpec((be, 128), lambda i: (i, 0))
    osh = jax.ShapeDtypeStruct((e_pad, 128), jnp.float32)
    return pl.pallas_call(
        body,
        grid=(e_pad // be,),
        in_specs=[
            blk, blk, blk,
            pl.BlockSpec((768, 128), lambda i: (0, 0)),
            pl.BlockSpec((8, 128), lambda i: (0, 0)),
        ],
        out_specs=[blk, blk],
        out_shape=[osh, osh],
        compiler_params=pltpu.CompilerParams(dimension_semantics=("parallel",)),
    )(gac, gb, e, wpack, bpack)


def _node_update(h, s0, s1, c0, c1, wpack, bpack):
    n = h.shape[0]
    bn = 2000

    def body(h_ref, s0_ref, s1_ref, c0_ref, c1_ref, w_ref, b_ref, out_ref):
        cnt = jnp.maximum(c0_ref[...] + c1_ref[...], 1.0)
        agg = (s0_ref[...] + s1_ref[...]) / cnt
        t = _leaky(
            _dot(h_ref[...], w_ref[0:128]) + _dot(agg, w_ref[128:256]) + b_ref[0:1, :]
        )
        t = _leaky(_dot(t, w_ref[256:384]) + b_ref[1:2, :])
        out_ref[...] = _dot(t, w_ref[384:512]) + b_ref[2:3, :]

    blk = pl.BlockSpec((bn, 128), lambda i: (i, 0))
    return pl.pallas_call(
        body,
        grid=(n // bn,),
        in_specs=[
            blk, blk, blk, blk, blk,
            pl.BlockSpec((512, 128), lambda i: (0, 0)),
            pl.BlockSpec((8, 128), lambda i: (0, 0)),
        ],
        out_specs=blk,
        out_shape=jax.ShapeDtypeStruct((n, 128), jnp.float32),
        compiler_params=pltpu.CompilerParams(dimension_semantics=("parallel",)),
    )(h, s0, s1, c0, c1, wpack, bpack)


def kernel(x, edge_index, edge_attr, params):
    n, d = x.shape
    e_num = edge_attr.shape[0]
    k = -(-e_num // (NW * CHUNK))
    e_pad = NW * CHUNK * k
    pad = e_pad - e_num

    row = edge_index[0].astype(jnp.int32)
    col = edge_index[1].astype(jnp.int32)
    row3 = jnp.pad(row, (0, pad)).reshape(NW, k, CHUNK)
    col3 = jnp.pad(col, (0, pad)).reshape(NW, k, CHUNK)
    e = jnp.pad(edge_attr, ((0, pad), (0, 0)))

    n_pad = NS * (-(-(-(-n // NS)) // 8) * 8)
    spread = max(n_pad - n, 1)
    pad_idx = n_pad - 1 - (jnp.arange(pad, dtype=jnp.int32) % spread)
    col3s = jnp.concatenate([col, pad_idx]).reshape(NW, k, CHUNK)

    cnt2 = _counts_fn(n, e_pad, k, e_num)(col3s)
    c0, c1 = cnt2[:n], cnt2[n_pad:n_pad + n]

    h = x
    for lp in params:
        (w1, b1), (w2, b2), (w3, b3) = lp["edge"]
        (v1, cb1), (v2, cb2), (v3, cb3) = lp["node1"]
        (u1, d1), (u2, d2), (u3, d3) = lp["node2"]

        wpre = jnp.concatenate([w1[0:128], w1[128:256], v1[0:128]], axis=0)
        hac, hb = _node_pre(h, wpre, _bias_pack([b1, cb1]))

        gac = _gather_fn(n, 128, e_pad, k)(hac, row3)
        gb = _gather_fn(n, 128, e_pad, k)(hb, col3)

        wedge = jnp.concatenate([w1[256:384], w2, w3, v1[128:256], v2, v3], axis=0)
        e, m = _edge_mlps(gac, gb, e, wedge, _bias_pack([b2, b3, cb2, cb3]), e_num)

        s2 = _scatter_fn(n, e_pad, k)(m, col3s)

        wn2 = jnp.concatenate([u1[0:128], u1[128:256], u2, u3], axis=0)
        h = _node_update(h, s2[:n], s2[n_pad:n_pad + n], c0, c1, wn2,
                         _bias_pack([d1, d2, d3]))

    return h


# speedup vs baseline: 1.3505x; 1.0418x over previous
import functools

import jax
import jax.numpy as jnp
from jax import lax
from jax.experimental import pallas as pl
from jax.experimental.pallas import tpu as pltpu
from jax.experimental.pallas import tpu_sc as plsc

NC = 2
NS = 16
NW = NC * NS
CHUNK = 128


def _leaky(t):
    return jnp.where(t >= 0, t, 0.01 * t)


def _dot(a, b):
    return jnp.dot(a, b, preferred_element_type=jnp.float32)


def _bias_pack(biases):
    rows = jnp.stack(biases, axis=0)
    return jnp.pad(rows, ((0, 8 - rows.shape[0]), (0, 0)))


def _pack_pair(lo, hi):
    lo16 = lax.bitcast_convert_type(lo.astype(jnp.bfloat16), jnp.uint16)
    hi16 = lax.bitcast_convert_type(hi.astype(jnp.bfloat16), jnp.uint16)
    word = lo16.astype(jnp.uint32) | (hi16.astype(jnp.uint32) << 16)
    return lax.bitcast_convert_type(word, jnp.float32)


def _unpack_pair(w):
    u = lax.bitcast_convert_type(w, jnp.uint32)
    f_lo = lax.bitcast_convert_type(u << 16, jnp.float32)
    f_hi = lax.bitcast_convert_type(u & jnp.uint32(0xFFFF0000), jnp.float32)
    return f_lo, f_hi


def _unpack_cols(w):
    f_lo, f_hi = _unpack_pair(w)
    return jnp.concatenate([f_lo, f_hi], axis=1)


@functools.lru_cache(maxsize=None)
def _gather_fn(n, f, e_pad, k, dtype=jnp.float32):
    ew = e_pad // NW
    mesh = plsc.VectorSubcoreMesh(core_axis_name="c", subcore_axis_name="s")

    @functools.partial(
        pl.kernel,
        mesh=mesh,
        out_type=jax.ShapeDtypeStruct((e_pad, f), dtype),
        scratch_types=[
            pltpu.VMEM((k, CHUNK), jnp.int32),
            pltpu.VMEM((2, CHUNK, f), dtype),
            pltpu.SemaphoreType.DMA,
            pltpu.SemaphoreType.DMA,
            pltpu.SemaphoreType.DMA,
        ],
    )
    def gather(table_hbm, idx_hbm, out_hbm, idx_v, rows_v, gsem, ws0, ws1):
        wid = lax.axis_index("s") * NC + lax.axis_index("c")
        base = wid * ew
        pltpu.sync_copy(idx_hbm.at[wid], idx_v)

        @pl.loop(0, k - k % 2, step=2)
        def _(j0):
            for b in range(2):
                j = j0 + b
                wsem = ws0 if b == 0 else ws1
                buf = rows_v.at[b]
                dst = out_hbm.at[pl.ds(base + j * CHUNK, CHUNK)]

                @pl.when(j >= 2)
                def _():
                    pltpu.make_async_copy(buf, dst, wsem).wait()

                pltpu.async_copy(table_hbm.at[idx_v.at[j]], buf, gsem)
                pltpu.make_async_copy(table_hbm.at[idx_v.at[j]], buf, gsem).wait()
                pltpu.async_copy(buf, dst, wsem)

        for b in range(2):
            wsem = ws0 if b == 0 else ws1
            pltpu.make_async_copy(
                rows_v.at[b],
                out_hbm.at[pl.ds(base + b * CHUNK, CHUNK)],
                wsem,
            ).wait()

        if k % 2:
            j = k - 1
            pltpu.async_copy(table_hbm.at[idx_v.at[j]], rows_v.at[0], gsem).wait()
            pltpu.sync_copy(rows_v.at[0], out_hbm.at[pl.ds(base + j * CHUNK, CHUNK)])

    return gather


@functools.lru_cache(maxsize=None)
def _scatter_fn(n, e_pad, k):
    ew = e_pad // NW
    zr = -(-(-(-n // NS)) // 8) * 8
    n_pad = NS * zr
    zfull, zrem = zr // CHUNK, zr % CHUNK
    mesh = plsc.VectorSubcoreMesh(core_axis_name="c", subcore_axis_name="s")

    def _zero_acc(rows_v, acc_sh, sid):
        zero = jnp.zeros((16,), jnp.float32)

        @pl.loop(0, CHUNK)
        def _(r):
            for c8 in range(8):
                rows_v[r, pl.ds(c8 * 16, 16)] = zero

        zb = sid * zr
        for t in range(zfull):
            pltpu.sync_copy(rows_v, acc_sh.at[pl.ds(zb + t * CHUNK, CHUNK)])
        if zrem:
            pltpu.sync_copy(
                rows_v.at[pl.ds(0, zrem)],
                acc_sh.at[pl.ds(zb + zfull * CHUNK, zrem)],
            )

    def _write_acc(rows_v, acc_sh, out_hbm, cid, sid):
        zb = sid * zr
        ob = cid * n_pad + zb
        for t in range(zfull):
            pltpu.sync_copy(acc_sh.at[pl.ds(zb + t * CHUNK, CHUNK)], rows_v)
            pltpu.sync_copy(rows_v, out_hbm.at[pl.ds(ob + t * CHUNK, CHUNK)])
        if zrem:
            pltpu.sync_copy(
                acc_sh.at[pl.ds(zb + zfull * CHUNK, zrem)],
                rows_v.at[pl.ds(0, zrem)],
            )
            pltpu.sync_copy(
                rows_v.at[pl.ds(0, zrem)],
                out_hbm.at[pl.ds(ob + zfull * CHUNK, zrem)],
            )

    @functools.partial(
        pl.kernel,
        mesh=mesh,
        out_type=jax.ShapeDtypeStruct((NC * n_pad, 128), jnp.float32),
        scratch_types=[
            pltpu.VMEM((k, CHUNK), jnp.int32),
            pltpu.VMEM((2, CHUNK, 128), jnp.float32),
            pltpu.VMEM_SHARED((n_pad, 128), jnp.float32),
            pltpu.SemaphoreType.DMA,
            pltpu.SemaphoreType.DMA,
            pltpu.SemaphoreType.DMA,
            pltpu.SemaphoreType.DMA,
        ],
    )
    def scatter(vals_hbm, idx_hbm, out_hbm, idx_v, rows_v, acc_sh,
                ls0, ls1, ss0, ss1):
        cid = lax.axis_index("c")
        sid = lax.axis_index("s")
        wid = sid * NC + cid
        base = wid * ew

        _zero_acc(rows_v.at[0], acc_sh, sid)
        plsc.subcore_barrier()

        pltpu.sync_copy(idx_hbm.at[wid], idx_v)

        @pl.loop(0, k - k % 2, step=2)
        def _(j0):
            for b in range(2):
                j = j0 + b
                lsem = ls0 if b == 0 else ls1
                ssem = ss0 if b == 0 else ss1
                buf = rows_v.at[b]
                src = vals_hbm.at[pl.ds(base + j * CHUNK, CHUNK)]

                @pl.when(j >= 2)
                def _():
                    pltpu.make_async_copy(buf, acc_sh.at[idx_v.at[j]], ssem).wait()

                pltpu.async_copy(src, buf, lsem)
                pltpu.make_async_copy(src, buf, lsem).wait()
                pltpu.async_copy(buf, acc_sh.at[idx_v.at[j]], ssem, add=True)

        for b in range(2):
            ssem = ss0 if b == 0 else ss1
            pltpu.make_async_copy(
                rows_v.at[b], acc_sh.at[idx_v.at[b]], ssem
            ).wait()

        if k % 2:
            pltpu.sync_copy(
                vals_hbm.at[pl.ds(base + (k - 1) * CHUNK, CHUNK)], rows_v.at[0]
            )
            pltpu.sync_copy(rows_v.at[0], acc_sh.at[idx_v.at[k - 1]], add=True)

        plsc.subcore_barrier()
        _write_acc(rows_v.at[0], acc_sh, out_hbm, cid, sid)

    return scatter


@functools.lru_cache(maxsize=None)
def _counts_fn(n, e_pad, k, e_num):
    ew = e_pad // NW
    zr = -(-(-(-n // NS)) // 8) * 8
    n_pad = NS * zr
    zfull, zrem = zr // CHUNK, zr % CHUNK
    mesh = plsc.VectorSubcoreMesh(core_axis_name="c", subcore_axis_name="s")

    @functools.partial(
        pl.kernel,
        mesh=mesh,
        out_type=jax.ShapeDtypeStruct((NC * n_pad, 128), jnp.float32),
        scratch_types=[
            pltpu.VMEM((k, CHUNK), jnp.int32),
            pltpu.VMEM((2, CHUNK, 128), jnp.float32),
            pltpu.VMEM_SHARED((n_pad, 128), jnp.float32),
            pltpu.SemaphoreType.DMA,
        ],
    )
    def counts(idx_hbm, out_hbm, idx_v, rows_v, acc_sh, sem):
        cid = lax.axis_index("c")
        sid = lax.axis_index("s")
        wid = sid * NC + cid
        base = wid * ew
        n_real = jnp.clip(e_num - base, 0, ew)
        kf = n_real // CHUNK
        prem = n_real % CHUNK

        zero = jnp.zeros((16,), jnp.float32)

        @pl.loop(0, CHUNK)
        def _(r):
            for c8 in range(8):
                rows_v[0, r, pl.ds(c8 * 16, 16)] = zero
                rows_v[1, r, pl.ds(c8 * 16, 16)] = jnp.where(
                    r < prem, 1.0, 0.0
                ) * jnp.ones((16,), jnp.float32)

        zb = sid * zr
        for t in range(zfull):
            pltpu.sync_copy(rows_v.at[0], acc_sh.at[pl.ds(zb + t * CHUNK, CHUNK)])
        if zrem:
            pltpu.sync_copy(
                rows_v.at[0, pl.ds(0, zrem)],
                acc_sh.at[pl.ds(zb + zfull * CHUNK, zrem)],
            )
        plsc.subcore_barrier()

        pltpu.sync_copy(idx_hbm.at[wid], idx_v)

        @pl.loop(0, CHUNK)
        def _(r):
            for c8 in range(8):
                rows_v[0, r, pl.ds(c8 * 16, 16)] = jnp.ones((16,), jnp.float32)

        @pl.loop(0, kf)
        def _(j):
            pltpu.sync_copy(rows_v.at[0], acc_sh.at[idx_v.at[j]], add=True)

        @pl.when(prem > 0)
        def _():
            pltpu.sync_copy(rows_v.at[1], acc_sh.at[idx_v.at[kf]], add=True)

        plsc.subcore_barrier()

        ob = cid * n_pad + zb
        for t in range(zfull):
            pltpu.sync_copy(acc_sh.at[pl.ds(zb + t * CHUNK, CHUNK)], rows_v.at[0])
            pltpu.sync_copy(rows_v.at[0], out_hbm.at[pl.ds(ob + t * CHUNK, CHUNK)])
        if zrem:
            pltpu.sync_copy(
                acc_sh.at[pl.ds(zb + zfull * CHUNK, zrem)],
                rows_v.at[0, pl.ds(0, zrem)],
            )
            pltpu.sync_copy(
                rows_v.at[0, pl.ds(0, zrem)],
                out_hbm.at[pl.ds(ob + zfull * CHUNK, zrem)],
            )

    return counts


def _node_pre(h, wpack, bpack):
    n = h.shape[0]
    bn = 2000

    def body(h_ref, w_ref, b_ref, hac_ref, hb_ref):
        hh = h_ref[...]
        ha = _dot(hh, w_ref[0:128]) + b_ref[0:1, :]
        hc = _dot(hh, w_ref[256:384]) + b_ref[1:2, :]
        hb = _dot(hh, w_ref[128:256])
        hac_ref[:, 0:64] = _pack_pair(ha[:, 0:64], ha[:, 64:128])
        hac_ref[:, 64:128] = _pack_pair(hc[:, 0:64], hc[:, 64:128])
        hb_ref[...] = hb

    return pl.pallas_call(
        body,
        grid=(n // bn,),
        in_specs=[
            pl.BlockSpec((bn, 128), lambda i: (i, 0)),
            pl.BlockSpec((384, 128), lambda i: (0, 0)),
            pl.BlockSpec((8, 128), lambda i: (0, 0)),
        ],
        out_specs=[
            pl.BlockSpec((bn, 128), lambda i: (i, 0)),
            pl.BlockSpec((bn, 128), lambda i: (i, 0)),
        ],
        out_shape=[
            jax.ShapeDtypeStruct((n, 128), jnp.float32),
            jax.ShapeDtypeStruct((n, 128), jnp.float32),
        ],
        compiler_params=pltpu.CompilerParams(dimension_semantics=("parallel",)),
    )(h, wpack, bpack)


def _edge_mlps(gac, gb, e, wpack, bpack, e_real):
    e_pad = e.shape[0]
    be = 4096

    def body(gac_ref, gb_ref, e_ref, w_ref, b_ref, enew_ref, m_ref):
        i = pl.program_id(0)
        ga = _unpack_cols(gac_ref[:, 0:64])
        gc = _unpack_cols(gac_ref[:, 64:128])
        u = _leaky(ga + gb_ref[...] + _dot(e_ref[...], w_ref[0:128]))
        u = _leaky(_dot(u, w_ref[128:256]) + b_ref[0:1, :])
        en = _dot(u, w_ref[256:384]) + b_ref[1:2, :]
        enew_ref[...] = en
        v = _leaky(gc + _dot(en, w_ref[384:512]))
        v = _leaky(_dot(v, w_ref[512:640]) + b_ref[2:3, :])
        m = _dot(v, w_ref[640:768]) + b_ref[3:4, :]
        rowid = i * be + lax.broadcasted_iota(jnp.int32, (be, 1), 0)
        m_ref[...] = jnp.where(rowid < e_real, m, 0.0)

    blk = pl.BlockSpec((be, 128), lambda i: (i, 0))
    osh = jax.ShapeDtypeStruct((e_pad, 128), jnp.float32)
    return pl.pallas_call(
        body,
        grid=(e_pad // be,),
        in_specs=[
            blk, blk, blk,
            pl.BlockSpec((768, 128), lambda i: (0, 0)),
            pl.BlockSpec((8, 128), lambda i: (0, 0)),
        ],
        out_specs=[blk, blk],
        out_shape=[osh, osh],
        compiler_params=pltpu.CompilerParams(dimension_semantics=("parallel",)),
    )(gac, gb, e, wpack, bpack)


def _node_update(h, s0, s1, c0, c1, wpack, bpack):
    n = h.shape[0]
    bn = 2000

    def body(h_ref, s0_ref, s1_ref, c0_ref, c1_ref, w_ref, b_ref, out_ref):
        cnt = jnp.maximum(c0_ref[...] + c1_ref[...], 1.0)
        agg = (s0_ref[...] + s1_ref[...]) / cnt
        t = _leaky(
            _dot(h_ref[...], w_ref[0:128]) + _dot(agg, w_ref[128:256]) + b_ref[0:1, :]
        )
        t = _leaky(_dot(t, w_ref[256:384]) + b_ref[1:2, :])
        out_ref[...] = _dot(t, w_ref[384:512]) + b_ref[2:3, :]

    blk = pl.BlockSpec((bn, 128), lambda i: (i, 0))
    return pl.pallas_call(
        body,
        grid=(n // bn,),
        in_specs=[
            blk, blk, blk, blk, blk,
            pl.BlockSpec((512, 128), lambda i: (0, 0)),
            pl.BlockSpec((8, 128), lambda i: (0, 0)),
        ],
        out_specs=blk,
        out_shape=jax.ShapeDtypeStruct((n, 128), jnp.float32),
        compiler_params=pltpu.CompilerParams(dimension_semantics=("parallel",)),
    )(h, s0, s1, c0, c1, wpack, bpack)


def kernel(x, edge_index, edge_attr, params):
    n, d = x.shape
    e_num = edge_attr.shape[0]
    k = -(-e_num // (NW * CHUNK))
    e_pad = NW * CHUNK * k
    pad = e_pad - e_num

    row = edge_index[0].astype(jnp.int32)
    col = edge_index[1].astype(jnp.int32)
    row3 = jnp.pad(row, (0, pad)).reshape(NW, k, CHUNK)
    col3 = jnp.pad(col, (0, pad)).reshape(NW, k, CHUNK)
    e = jnp.pad(edge_attr, ((0, pad), (0, 0)))

    n_pad = NS * (-(-(-(-n // NS)) // 8) * 8)
    spread = max(n_pad - n, 1)
    pad_idx = n_pad - 1 - (jnp.arange(pad, dtype=jnp.int32) % spread)
    col3s = jnp.concatenate([col, pad_idx]).reshape(NW, k, CHUNK)

    cnt2 = _counts_fn(n, e_pad, k, e_num)(col3s)
    c0, c1 = cnt2[:n], cnt2[n_pad:n_pad + n]

    h = x
    for lp in params:
        (w1, b1), (w2, b2), (w3, b3) = lp["edge"]
        (v1, cb1), (v2, cb2), (v3, cb3) = lp["node1"]
        (u1, d1), (u2, d2), (u3, d3) = lp["node2"]

        wpre = jnp.concatenate([w1[0:128], w1[128:256], v1[0:128]], axis=0)
        hac, hb = _node_pre(h, wpre, _bias_pack([b1, cb1]))

        gac = _gather_fn(n, 128, e_pad, k)(hac, row3)
        gb = _gather_fn(n, 128, e_pad, k)(hb, col3)

        wedge = jnp.concatenate([w1[256:384], w2, w3, v1[128:256], v2, v3], axis=0)
        e, m = _edge_mlps(gac, gb, e, wedge, _bias_pack([b2, b3, cb2, cb3]), e_num)

        s2 = _scatter_fn(n, e_pad, k)(m, col3s)

        wn2 = jnp.concatenate([u1[0:128], u1[128:256], u2, u3], axis=0)
        h = _node_update(h, s2[:n], s2[n_pad:n_pad + n], c0, c1, wn2,
                         _bias_pack([d1, d2, d3]))

    return h
